# Initial kernel scaffold; baseline (speedup 1.0000x reference)
import jax
import jax.numpy as jnp
from jax.experimental import pallas as pl


def kernel(x, edge_index, batch, W_rel1, b_rel1, W_root1, W_rel2, b_rel2, W_root2, W_rel3, b_rel3, W_root3, W_lin, b_lin):
    raise NotImplementedError("write your pallas kernel here")


# speedup vs baseline: 2.5132x; 2.5132x over previous
import functools

import jax
import jax.numpy as jnp
from jax import lax
from jax.experimental import pallas as pl
from jax.experimental.pallas import tpu as pltpu
from jax.experimental.pallas import tpu_sc as plsc

N = 10000
E = 160000
F = 256
C = 64
G = 128

TILES = 16
HALF = F // 2
CH = 128
PT = 10240
EPAD = PT * TILES
NCHUNK = PT // CH
RPT = 624
RLAST = N - 15 * RPT + 8
BLK = 1000
GRID = N // BLK

_mesh = plsc.VectorSubcoreMesh(core_axis_name="c", subcore_axis_name="s")


@functools.partial(
    pl.kernel,
    out_type=[
        jax.ShapeDtypeStruct((N, HALF), jnp.float32),
        jax.ShapeDtypeStruct((N, HALF), jnp.float32),
    ],
    mesh=_mesh,
    scratch_types=[
        pltpu.VMEM((CH,), jnp.int32),
        pltpu.VMEM((CH,), jnp.int32),
        pltpu.VMEM((CH, HALF), jnp.float32),
        pltpu.SemaphoreType.DMA,
        pltpu.VMEM_SHARED((N + 8, HALF), jnp.float32),
    ],
)
def _sc_agg(yA, yB, src, dst, zeros, outA, outB, idx_s, idx_d, rows, sem,
            acc):
    c = lax.axis_index("c")
    s = lax.axis_index("s")
    row0 = pl.multiple_of(s * RPT, 8)
    @pl.when(s < TILES - 1)
    def _():
        pltpu.sync_copy(zeros.at[pl.ds(0, RPT)], acc.at[pl.ds(row0, RPT)])

    @pl.when(s == TILES - 1)
    def _():
        pltpu.sync_copy(zeros, acc.at[pl.ds(15 * RPT, RLAST)])

    plsc.subcore_barrier()

    def edge_loop(y_ref):
        def body(j, carry):
            off = pl.multiple_of(s * PT + j * CH, CH)
            pltpu.sync_copy(src.at[pl.ds(off, CH)], idx_s)
            pltpu.sync_copy(dst.at[pl.ds(off, CH)], idx_d)
            pltpu.async_copy(y_ref.at[idx_s], rows, sem).wait()
            pltpu.sync_copy(rows, acc.at[idx_d], add=True)
            return carry
        lax.fori_loop(0, NCHUNK, body, 0)

    @pl.when(c == 0)
    def _():
        edge_loop(yA)

    @pl.when(c == 1)
    def _():
        edge_loop(yB)

    plsc.subcore_barrier()

    def copy_out(out_ref):
        @pl.when(s < TILES - 1)
        def _():
            pltpu.sync_copy(acc.at[pl.ds(row0, RPT)],
                            out_ref.at[pl.ds(row0, RPT)])

        @pl.when(s == TILES - 1)
        def _():
            pltpu.sync_copy(acc.at[pl.ds(15 * RPT, RLAST - 8)],
                            out_ref.at[pl.ds(15 * RPT, RLAST - 8)])

    @pl.when(c == 0)
    def _():
        copy_out(outA)

    @pl.when(c == 1)
    def _():
        copy_out(outB)


def _mm_in_body(x_ref, wrel_ref, wroot_ref, b_ref, yA_ref, yB_ref, r_ref):
    xb = x_ref[...]
    y = jnp.dot(xb, wrel_ref[...], preferred_element_type=jnp.float32)
    r = jnp.dot(xb, wroot_ref[...], preferred_element_type=jnp.float32)
    yA_ref[...] = y[:, :HALF]
    yB_ref[...] = y[:, HALF:]
    r_ref[...] = r + b_ref[...]


def _mm_mid_body(aA_ref, aB_ref, rin_ref, wrel_ref, wroot_ref, b_ref,
                 yA_ref, yB_ref, r_ref):
    agg = jnp.concatenate([aA_ref[...], aB_ref[...]], axis=1)
    t = jnp.maximum(agg + rin_ref[...], 0.0)
    y = jnp.dot(t, wrel_ref[...], preferred_element_type=jnp.float32)
    r = jnp.dot(t, wroot_ref[...], preferred_element_type=jnp.float32)
    yA_ref[...] = y[:, :HALF]
    yB_ref[...] = y[:, HALF:]
    r_ref[...] = r + b_ref[...]


def _pool_body(aA_ref, aB_ref, rin_ref, bat_ref, wlin_ref, blin_ref,
               out_ref, pooled, cnt):
    i = pl.program_id(0)
    h = jnp.concatenate([aA_ref[...], aB_ref[...]], axis=1) + rin_ref[...]
    bat = bat_ref[0, 0, :]
    oneh = (bat[:, None] == lax.broadcasted_iota(jnp.int32, (BLK, G), 1)
            ).astype(jnp.float32)

    @pl.when(i == 0)
    def _():
        pooled[...] = jnp.zeros_like(pooled)
        cnt[...] = jnp.zeros_like(cnt)

    dn = (((0,), (0,)), ((), ()))
    pooled[...] += lax.dot_general(oneh, h, dimension_numbers=dn,
                                   preferred_element_type=jnp.float32)
    cnt[...] += lax.dot_general(oneh, jnp.ones_like(h), dimension_numbers=dn,
                                preferred_element_type=jnp.float32)

    @pl.when(i == pl.num_programs(0) - 1)
    def _():
        pm = pooled[...] / jnp.maximum(cnt[...], 1.0)
        out_ref[...] = (jnp.dot(pm, wlin_ref[...],
                                preferred_element_type=jnp.float32)
                        + blin_ref[...])


_W_SPEC = pl.BlockSpec((F, F), lambda i: (0, 0))
_B_SPEC = pl.BlockSpec((1, F), lambda i: (0, 0))
_ROW_SPEC = pl.BlockSpec((BLK, F), lambda i: (i, 0))
_HALF_SPEC = pl.BlockSpec((BLK, HALF), lambda i: (i, 0))

_Y_SHAPES = [
    jax.ShapeDtypeStruct((N, HALF), jnp.float32),
    jax.ShapeDtypeStruct((N, HALF), jnp.float32),
    jax.ShapeDtypeStruct((N, F), jnp.float32),
]

_mm_in = pl.pallas_call(
    _mm_in_body,
    grid=(GRID,),
    in_specs=[_ROW_SPEC, _W_SPEC, _W_SPEC, _B_SPEC],
    out_specs=[_HALF_SPEC, _HALF_SPEC, _ROW_SPEC],
    out_shape=_Y_SHAPES,
)

_mm_mid = pl.pallas_call(
    _mm_mid_body,
    grid=(GRID,),
    in_specs=[_HALF_SPEC, _HALF_SPEC, _ROW_SPEC, _W_SPEC, _W_SPEC, _B_SPEC],
    out_specs=[_HALF_SPEC, _HALF_SPEC, _ROW_SPEC],
    out_shape=_Y_SHAPES,
)

_pool = pl.pallas_call(
    _pool_body,
    grid=(GRID,),
    in_specs=[
        _HALF_SPEC, _HALF_SPEC, _ROW_SPEC,
        pl.BlockSpec((1, 1, BLK), lambda i: (i, 0, 0)),
        pl.BlockSpec((F, C), lambda i: (0, 0)),
        pl.BlockSpec((1, C), lambda i: (0, 0)),
    ],
    out_specs=pl.BlockSpec((G, C), lambda i: (0, 0)),
    out_shape=jax.ShapeDtypeStruct((G, C), jnp.float32),
    scratch_shapes=[
        pltpu.VMEM((G, F), jnp.float32),
        pltpu.VMEM((G, F), jnp.float32),
    ],
)


def kernel(x, edge_index, batch, W_rel1, b_rel1, W_root1, W_rel2, b_rel2,
           W_root2, W_rel3, b_rel3, W_root3, W_lin, b_lin):
    src = edge_index[0]
    dst = edge_index[1]
    pad = EPAD - E
    src_p = jnp.concatenate([src, jnp.zeros((pad,), src.dtype)])
    dst_p = jnp.concatenate([dst, jnp.full((pad,), N, dst.dtype)])
    zeros = jnp.zeros((RLAST, HALF), jnp.float32)
    bat3 = batch.reshape(GRID, 1, BLK)

    yA, yB, r = _mm_in(x, W_rel1, W_root1, b_rel1.reshape(1, F))
    aA, aB = _sc_agg(yA, yB, src_p, dst_p, zeros)
    yA, yB, r = _mm_mid(aA, aB, r, W_rel2, W_root2, b_rel2.reshape(1, F))
    aA, aB = _sc_agg(yA, yB, src_p, dst_p, zeros)
    yA, yB, r = _mm_mid(aA, aB, r, W_rel3, W_root3, b_rel3.reshape(1, F))
    aA, aB = _sc_agg(yA, yB, src_p, dst_p, zeros)
    out = _pool(aA, aB, r, bat3, W_lin, b_lin.reshape(1, C))
    return out


# speedup vs baseline: 3.4517x; 1.3735x over previous
import functools

import jax
import jax.numpy as jnp
from jax import lax
from jax.experimental import pallas as pl
from jax.experimental.pallas import tpu as pltpu
from jax.experimental.pallas import tpu_sc as plsc

N = 10000
E = 160000
F = 256
C = 64
G = 128

TILES = 16
HALF = F // 2
CH = 128
PT = 10240
EPAD = PT * TILES
NCHUNK = PT // CH
RPT = 624
RLAST = N - 15 * RPT + 8
BLK = 1000
GRID = N // BLK

_mesh = plsc.VectorSubcoreMesh(core_axis_name="c", subcore_axis_name="s")


@functools.partial(
    pl.kernel,
    out_type=[
        jax.ShapeDtypeStruct((N, HALF), jnp.float32),
        jax.ShapeDtypeStruct((N, HALF), jnp.float32),
    ],
    mesh=_mesh,
    scratch_types=[
        pltpu.VMEM((NCHUNK // 2, CH), jnp.int32),
        pltpu.VMEM((NCHUNK // 2, CH), jnp.int32),
        pltpu.VMEM((CH, HALF), jnp.float32),
        pltpu.VMEM((CH, HALF), jnp.float32),
        pltpu.SemaphoreType.DMA,
        pltpu.SemaphoreType.DMA,
        pltpu.VMEM_SHARED((N + 8, HALF), jnp.float32),
    ],
)
def _sc_agg(yA, yB, src, dst, zeros, outA, outB, idx_s, idx_d, rows0, rows1,
            sem0, sem1, acc):
    c = lax.axis_index("c")
    s = lax.axis_index("s")
    row0 = pl.multiple_of(s * RPT, 8)
    @pl.when(s < TILES - 1)
    def _():
        pltpu.sync_copy(zeros.at[pl.ds(0, RPT)], acc.at[pl.ds(row0, RPT)])

    @pl.when(s == TILES - 1)
    def _():
        pltpu.sync_copy(zeros, acc.at[pl.ds(15 * RPT, RLAST)])

    plsc.subcore_barrier()

    def edge_loop(y_ref):
        hch = NCHUNK // 2
        for h in range(2):
            crow = pl.multiple_of(s * NCHUNK + h * hch, 8)
            pltpu.sync_copy(src.at[pl.ds(crow, hch)], idx_s)
            pltpu.sync_copy(dst.at[pl.ds(crow, hch)], idx_d)
            pltpu.async_copy(y_ref.at[idx_s.at[0]], rows0, sem0)

            def body(k, carry):
                j0 = k * 2
                pltpu.make_async_copy(y_ref.at[idx_s.at[j0]], rows0,
                                      sem0).wait()
                pltpu.async_copy(y_ref.at[idx_s.at[j0 + 1]], rows1, sem1)
                pltpu.sync_copy(rows0, acc.at[idx_d.at[j0]], add=True)
                pltpu.make_async_copy(y_ref.at[idx_s.at[j0 + 1]], rows1,
                                      sem1).wait()

                @pl.when(k < hch // 2 - 1)
                def _():
                    pltpu.async_copy(y_ref.at[idx_s.at[j0 + 2]], rows0, sem0)

                pltpu.sync_copy(rows1, acc.at[idx_d.at[j0 + 1]], add=True)
                return carry
            lax.fori_loop(0, hch // 2, body, 0)

    @pl.when(c == 0)
    def _():
        edge_loop(yA)

    @pl.when(c == 1)
    def _():
        edge_loop(yB)

    plsc.subcore_barrier()

    def copy_out(out_ref):
        @pl.when(s < TILES - 1)
        def _():
            pltpu.sync_copy(acc.at[pl.ds(row0, RPT)],
                            out_ref.at[pl.ds(row0, RPT)])

        @pl.when(s == TILES - 1)
        def _():
            pltpu.sync_copy(acc.at[pl.ds(15 * RPT, RLAST - 8)],
                            out_ref.at[pl.ds(15 * RPT, RLAST - 8)])

    @pl.when(c == 0)
    def _():
        copy_out(outA)

    @pl.when(c == 1)
    def _():
        copy_out(outB)


def _mm_in_body(x_ref, wrel_ref, wroot_ref, b_ref, yA_ref, yB_ref, r_ref):
    xb = x_ref[...]
    y = jnp.dot(xb, wrel_ref[...], preferred_element_type=jnp.float32)
    r = jnp.dot(xb, wroot_ref[...], preferred_element_type=jnp.float32)
    yA_ref[...] = y[:, :HALF]
    yB_ref[...] = y[:, HALF:]
    r_ref[...] = r + b_ref[...]


def _mm_mid_body(aA_ref, aB_ref, rin_ref, wrel_ref, wroot_ref, b_ref,
                 yA_ref, yB_ref, r_ref):
    agg = jnp.concatenate([aA_ref[...], aB_ref[...]], axis=1)
    t = jnp.maximum(agg + rin_ref[...], 0.0)
    y = jnp.dot(t, wrel_ref[...], preferred_element_type=jnp.float32)
    r = jnp.dot(t, wroot_ref[...], preferred_element_type=jnp.float32)
    yA_ref[...] = y[:, :HALF]
    yB_ref[...] = y[:, HALF:]
    r_ref[...] = r + b_ref[...]


def _pool_body(aA_ref, aB_ref, rin_ref, bat_ref, wlin_ref, blin_ref,
               out_ref, pooled, cnt):
    i = pl.program_id(0)
    h = jnp.concatenate([aA_ref[...], aB_ref[...]], axis=1) + rin_ref[...]
    bat = bat_ref[0, 0, :]
    oneh = (bat[:, None] == lax.broadcasted_iota(jnp.int32, (BLK, G), 1)
            ).astype(jnp.float32)

    @pl.when(i == 0)
    def _():
        pooled[...] = jnp.zeros_like(pooled)
        cnt[...] = jnp.zeros_like(cnt)

    dn = (((0,), (0,)), ((), ()))
    pooled[...] += lax.dot_general(oneh, h, dimension_numbers=dn,
                                   preferred_element_type=jnp.float32)
    cnt[...] += lax.dot_general(oneh, jnp.ones_like(h), dimension_numbers=dn,
                                preferred_element_type=jnp.float32)

    @pl.when(i == pl.num_programs(0) - 1)
    def _():
        pm = pooled[...] / jnp.maximum(cnt[...], 1.0)
        out_ref[...] = (jnp.dot(pm, wlin_ref[...],
                                preferred_element_type=jnp.float32)
                        + blin_ref[...])


_W_SPEC = pl.BlockSpec((F, F), lambda i: (0, 0))
_B_SPEC = pl.BlockSpec((1, F), lambda i: (0, 0))
_ROW_SPEC = pl.BlockSpec((BLK, F), lambda i: (i, 0))
_HALF_SPEC = pl.BlockSpec((BLK, HALF), lambda i: (i, 0))

_Y_SHAPES = [
    jax.ShapeDtypeStruct((N, HALF), jnp.float32),
    jax.ShapeDtypeStruct((N, HALF), jnp.float32),
    jax.ShapeDtypeStruct((N, F), jnp.float32),
]

_mm_in = pl.pallas_call(
    _mm_in_body,
    grid=(GRID,),
    in_specs=[_ROW_SPEC, _W_SPEC, _W_SPEC, _B_SPEC],
    out_specs=[_HALF_SPEC, _HALF_SPEC, _ROW_SPEC],
    out_shape=_Y_SHAPES,
)

_mm_mid = pl.pallas_call(
    _mm_mid_body,
    grid=(GRID,),
    in_specs=[_HALF_SPEC, _HALF_SPEC, _ROW_SPEC, _W_SPEC, _W_SPEC, _B_SPEC],
    out_specs=[_HALF_SPEC, _HALF_SPEC, _ROW_SPEC],
    out_shape=_Y_SHAPES,
)

_pool = pl.pallas_call(
    _pool_body,
    grid=(GRID,),
    in_specs=[
        _HALF_SPEC, _HALF_SPEC, _ROW_SPEC,
        pl.BlockSpec((1, 1, BLK), lambda i: (i, 0, 0)),
        pl.BlockSpec((F, C), lambda i: (0, 0)),
        pl.BlockSpec((1, C), lambda i: (0, 0)),
    ],
    out_specs=pl.BlockSpec((G, C), lambda i: (0, 0)),
    out_shape=jax.ShapeDtypeStruct((G, C), jnp.float32),
    scratch_shapes=[
        pltpu.VMEM((G, F), jnp.float32),
        pltpu.VMEM((G, F), jnp.float32),
    ],
)


def kernel(x, edge_index, batch, W_rel1, b_rel1, W_root1, W_rel2, b_rel2,
           W_root2, W_rel3, b_rel3, W_root3, W_lin, b_lin):
    src = edge_index[0]
    dst = edge_index[1]
    pad = EPAD - E
    src_p = jnp.concatenate([src, jnp.zeros((pad,), src.dtype)])
    src_p = src_p.reshape(EPAD // CH, CH)
    dst_p = jnp.concatenate([dst, jnp.full((pad,), N, dst.dtype)])
    dst_p = dst_p.reshape(EPAD // CH, CH)
    zeros = jnp.zeros((RLAST, HALF), jnp.float32)
    bat3 = batch.reshape(GRID, 1, BLK)

    yA, yB, r = _mm_in(x, W_rel1, W_root1, b_rel1.reshape(1, F))
    aA, aB = _sc_agg(yA, yB, src_p, dst_p, zeros)
    yA, yB, r = _mm_mid(aA, aB, r, W_rel2, W_root2, b_rel2.reshape(1, F))
    aA, aB = _sc_agg(yA, yB, src_p, dst_p, zeros)
    yA, yB, r = _mm_mid(aA, aB, r, W_rel3, W_root3, b_rel3.reshape(1, F))
    aA, aB = _sc_agg(yA, yB, src_p, dst_p, zeros)
    out = _pool(aA, aB, r, bat3, W_lin, b_lin.reshape(1, C))
    return out
